# baseline (speedup 1.0000x reference)
import functools

import jax
import jax.numpy as jnp
from jax import lax
from jax.experimental import pallas as pl
from jax.experimental.pallas import tpu as pltpu
from jax.experimental.pallas import tpu_sc as plsc

_N = 50000
_C = 64
_K = 16
_G = 8
_W = 128

_NC = 2
_NS = 16
_NW = _NC * _NS
_CH = 256
_NIDX = _N * _K
_NCHUNK0 = 2
_DEPTH = 2
_NIDX_C = _NIDX // _NCHUNK0
_NCHW_C0 = -(-_NIDX_C // (_NW * _CH * _DEPTH)) * _DEPTH
_NPC0 = _NW * _CH * _NCHW_C0
_NPAD = _NCHUNK0 * _NPC0


_NCHUNK = _NCHUNK0
_NCHW_C = _NCHW_C0
_NPC = _NPC0


def _sc_gather_chunk_body(chunk_off, tbl_ref, idx_ref, gx_ref, *scr):
    w = lax.axis_index("s") * _NC + lax.axis_index("c")
    idx_v = scr[:_DEPTH]
    rows_v = scr[_DEPTH:2 * _DEPTH]
    gsem = scr[2 * _DEPTH:3 * _DEPTH]
    ssem = scr[3 * _DEPTH:4 * _DEPTH]

    def base(g):
        return (g * _NW + w) * _CH

    def load_and_fire(g, b):
        pltpu.sync_copy(idx_ref.at[pl.ds(chunk_off + base(g), _CH)], idx_v[b])
        pltpu.async_copy(tbl_ref.at[idx_v[b]], rows_v[b], gsem[b])

    def wait_gather(b):
        pltpu.make_async_copy(tbl_ref.at[idx_v[b]], rows_v[b], gsem[b]).wait()

    def start_store(g, b):
        pltpu.async_copy(rows_v[b], gx_ref.at[pl.ds(base(g), _CH), :], ssem[b])

    def wait_store(b):
        pltpu.make_async_copy(rows_v[b], gx_ref.at[pl.ds(0, _CH), :], ssem[b]).wait()

    for b in range(_DEPTH):
        load_and_fire(b, b)

    def step(t, carry):
        g0 = _DEPTH * t
        for b in range(_DEPTH):
            g = g0 + b
            wait_gather(b)
            start_store(g, b)

            @pl.when(g + _DEPTH < _NCHW_C)
            def _(b=b, g=g):
                wait_store(b)
                load_and_fire(g + _DEPTH, b)

        return carry

    lax.fori_loop(0, _NCHW_C // _DEPTH, step, 0)
    for b in range(_DEPTH):
        wait_store(b)


def _make_sc_gather(chunk):
    body = functools.partial(_sc_gather_chunk_body, chunk * _NPC)
    return functools.partial(
        pl.kernel,
        out_type=jax.ShapeDtypeStruct((_NPC, _W), jnp.float32),
        mesh=plsc.VectorSubcoreMesh(core_axis_name="c", subcore_axis_name="s"),
        scratch_types=(
            [pltpu.VMEM((_CH,), jnp.int32) for _ in range(_DEPTH)]
            + [pltpu.VMEM((_CH, _W), jnp.float32) for _ in range(_DEPTH)]
            + [pltpu.SemaphoreType.DMA for _ in range(2 * _DEPTH)]
        ),
    )(body)


def _tc_body(g_ref, t_ref,
             wkv_ref, p2w_ref, q2w_ref, qb_ref, t1w_ref, lo_ref,
             mv_ref, mk_ref, p1m_ref, pfs_ref, pft_ref,
             w1m_ref, w2s_ref, w2t_ref, w2m_ref, tile2_ref,
             out_ref):
    f32 = jnp.float32
    B = out_ref.shape[0]
    BK = B * _K

    g = g_ref[...]
    tb = t_ref[...]
    q2 = jnp.dot(tb, q2w_ref[...], preferred_element_type=f32) + qb_ref[...]

    aj = jnp.dot(g, p1m_ref[...], preferred_element_type=f32)
    bi = jnp.dot(tb, p1m_ref[...], preferred_element_type=f32)
    dt = (aj.reshape(B, _K, 8) - bi.reshape(B, 1, 8)).reshape(BK, 8)
    dt = jnp.maximum(dt * pfs_ref[...] + pft_ref[...], 0.0)
    pr2 = jnp.dot(dt, p2w_ref[...], preferred_element_type=f32)

    gkv = jnp.dot(g, wkv_ref[...], preferred_element_type=f32)
    u = ((gkv + pr2).reshape(B, _K, _W) - q2.reshape(B, 1, _W)).reshape(BK, _W)
    a = jnp.maximum(u + t1w_ref[...], lo_ref[...])

    w8 = jnp.dot(a, w1m_ref[...], preferred_element_type=f32)
    w8 = jnp.maximum(w8 * w2s_ref[...] + w2t_ref[...], 0.0)
    w8 = jnp.dot(w8, w2m_ref[...], preferred_element_type=f32)
    e = jnp.exp(w8)
    ew = jnp.dot(e, tile2_ref[...], preferred_element_type=f32)

    m = a * mv_ref[...] + mk_ref[...]
    y = m * ew
    ys = jnp.sum(y.reshape(B, _K, _W), axis=1)
    out_ref[...] = ys[:, _C:] / ys[:, :_C]


_B = 200
_NQC = _N // _NCHUNK
_GRID = _NQC // _B


def _tc_call(chunk, g, tbl, consts):
    goff = chunk * _GRID
    full = lambda shp: pl.BlockSpec(shp, lambda i: (0, 0))
    in_specs = [
        pl.BlockSpec((_B * _K, _W), lambda i: (i, 0)),
        pl.BlockSpec((_B, _W), lambda i: (i + goff, 0)),
    ] + [full(c.shape) for c in consts]
    return pl.pallas_call(
        _tc_body,
        grid=(_GRID,),
        in_specs=in_specs,
        out_specs=pl.BlockSpec((_B, _C), lambda i: (i, 0)),
        out_shape=jax.ShapeDtypeStruct((_NQC, _C), jnp.float32),
        compiler_params=pltpu.CompilerParams(
            dimension_semantics=("arbitrary",),
        ),
    )(g, tbl, *consts)


def kernel(p, x, idx, Wq, bq, Wk, bk, Wv, bv, p1W, p1b, pbn_g, pbn_b, pbn_m,
           pbn_v, p2W, p2b, wbn1_g, wbn1_b, wbn1_m, wbn1_v, w1W, w1b, wbn2_g,
           wbn2_b, wbn2_m, wbn2_v, w2W, w2b):
    f32 = jnp.float32
    idx32 = idx.reshape(-1).astype(jnp.int32)
    idxp = jnp.zeros((_NPAD,), jnp.int32)
    for c in range(_NCHUNK):
        idxp = idxp.at[c * _NPC : c * _NPC + _NIDX_C].set(
            idx32[c * _NIDX_C : (c + 1) * _NIDX_C])
    tbl = jnp.concatenate(
        [x, p, jnp.zeros((_N, _W - _C - 3), f32)], axis=1)

    gs = [_make_sc_gather(c)(tbl, idxp) for c in range(_NCHUNK)]

    eps = 1e-5
    s1 = wbn1_g / jnp.sqrt(wbn1_v + eps)
    t1 = wbn1_b - wbn1_m * s1
    s2 = wbn2_g / jnp.sqrt(wbn2_v + eps)
    t2 = wbn2_b - wbn2_m * s2
    ps = pbn_g / jnp.sqrt(pbn_v + eps)
    pt = pbn_b - pbn_m * ps

    def pad8(v):
        return jnp.zeros((1, 8), f32).at[0, : v.shape[0]].set(v)

    z64 = jnp.zeros((64,), f32)
    one64 = jnp.ones((64,), f32)

    wkv = (jnp.zeros((_W, _W), f32)
           .at[:_C, :_C].set(Wk.T * s1[None, :])
           .at[:_C, _C:].set(Wv.T))
    p2w = jnp.zeros((8, _W), f32).at[:3, :_C].set(p2W.T * s1[None, :]) \
                                  .at[:3, _C:].set(p2W.T)
    q2w = jnp.zeros((_W, _W), f32).at[:_C, :_C].set(Wq.T * s1[None, :])
    qb = jnp.concatenate([(bq - bk - p2b) * s1, -(bv + p2b)]).reshape(1, _W)
    t1w = jnp.concatenate([t1, z64]).reshape(1, _W)
    lo = jnp.concatenate([z64, jnp.full((64,), -jnp.inf, f32)]).reshape(1, _W)
    mv = jnp.concatenate([z64, one64]).reshape(1, _W)
    mk = jnp.concatenate([one64, z64]).reshape(1, _W)
    p1m = jnp.zeros((_W, 8), f32).at[_C:_C + 3, :3].set(p1W.T)
    pfs = pad8(ps)
    pft = pad8(p1b * ps + pt)
    w1m = jnp.zeros((_W, 8), f32).at[:_C, :].set(w1W.T)
    w2sp = pad8(s2)
    w2tp = pad8(s2 * w1b + t2)
    tile = jnp.tile(jnp.eye(_G, dtype=f32), (1, _C // _G))
    tile2 = jnp.concatenate([tile, tile], axis=1)

    consts = [
        wkv, p2w, q2w, qb, t1w, lo, mv, mk,
        p1m, pfs, pft, w1m, w2sp, w2tp, w2W.T, tile2,
    ]
    outs = [_tc_call(c, gs[c], tbl, consts) for c in range(_NCHUNK)]
    return jnp.concatenate(outs, axis=0)


# speedup vs baseline: 1.8034x; 1.8034x over previous
import functools

import jax
import jax.numpy as jnp
from jax import lax
from jax.experimental import pallas as pl
from jax.experimental.pallas import tpu as pltpu
from jax.experimental.pallas import tpu_sc as plsc

_N = 50000
_C = 64
_K = 16
_G = 8
_W = 128

_NC = 2
_NS = 16
_NW = _NC * _NS
_CH = 128
_NIDX = _N * _K
_NCHUNK0 = 2
_DEPTH = 2
_NIDX_C = _NIDX // _NCHUNK0
_NCHW_C0 = -(-_NIDX_C // (_NW * _CH * _DEPTH)) * _DEPTH
_NPC0 = _NW * _CH * _NCHW_C0
_NPAD = _NCHUNK0 * _NPC0


_NCHUNK = _NCHUNK0
_NCHW_C = _NCHW_C0
_NPC = _NPC0


def _sc_gather_chunk_body(chunk_off, tbl_ref, idx_ref, gx_ref, *scr):
    w = lax.axis_index("s") * _NC + lax.axis_index("c")
    idx_v = scr[:_DEPTH]
    rows_v = scr[_DEPTH:2 * _DEPTH]
    gsem = scr[2 * _DEPTH:3 * _DEPTH]
    ssem = scr[3 * _DEPTH:4 * _DEPTH]

    def base(g):
        return (g * _NW + w) * _CH

    def load_and_fire(g, b):
        pltpu.sync_copy(idx_ref.at[pl.ds(chunk_off + base(g), _CH)], idx_v[b])
        pltpu.async_copy(tbl_ref.at[idx_v[b]], rows_v[b], gsem[b])

    def wait_gather(b):
        pltpu.make_async_copy(tbl_ref.at[idx_v[b]], rows_v[b], gsem[b]).wait()

    def start_store(g, b):
        pltpu.async_copy(rows_v[b], gx_ref.at[pl.ds(base(g), _CH), :], ssem[b])

    def wait_store(b):
        pltpu.make_async_copy(rows_v[b], gx_ref.at[pl.ds(0, _CH), :], ssem[b]).wait()

    for b in range(_DEPTH):
        load_and_fire(b, b)

    def step(t, carry):
        g0 = _DEPTH * t
        for b in range(_DEPTH):
            g = g0 + b
            wait_gather(b)
            start_store(g, b)

            @pl.when(g + _DEPTH < _NCHW_C)
            def _(b=b, g=g):
                wait_store(b)
                load_and_fire(g + _DEPTH, b)

        return carry

    lax.fori_loop(0, _NCHW_C // _DEPTH, step, 0)
    for b in range(_DEPTH):
        wait_store(b)


def _make_sc_gather(chunk):
    body = functools.partial(_sc_gather_chunk_body, chunk * _NPC)
    return functools.partial(
        pl.kernel,
        out_type=jax.ShapeDtypeStruct((_NPC, _W), jnp.float32),
        mesh=plsc.VectorSubcoreMesh(core_axis_name="c", subcore_axis_name="s"),
        scratch_types=(
            [pltpu.VMEM((_CH,), jnp.int32) for _ in range(_DEPTH)]
            + [pltpu.VMEM((_CH, _W), jnp.float32) for _ in range(_DEPTH)]
            + [pltpu.SemaphoreType.DMA for _ in range(2 * _DEPTH)]
        ),
    )(body)


def _tc_body(g_ref, t_ref,
             wkv_ref, p2w_ref, q2w_ref, qb_ref, t1w_ref, lo_ref,
             mv_ref, mk_ref, p1m_ref, pfs_ref, pft_ref,
             w1m_ref, w2s_ref, w2t_ref, w2m_ref, tile2_ref,
             out_ref):
    f32 = jnp.float32
    B = out_ref.shape[0]
    BK = B * _K

    g = g_ref[...]
    tb = t_ref[...]
    q2 = jnp.dot(tb, q2w_ref[...], preferred_element_type=f32) + qb_ref[...]

    aj = jnp.dot(g, p1m_ref[...], preferred_element_type=f32)
    bi = jnp.dot(tb, p1m_ref[...], preferred_element_type=f32)
    dt = (aj.reshape(B, _K, 8) - bi.reshape(B, 1, 8)).reshape(BK, 8)
    dt = jnp.maximum(dt * pfs_ref[...] + pft_ref[...], 0.0)
    pr2 = jnp.dot(dt, p2w_ref[...], preferred_element_type=f32)

    gkv = jnp.dot(g, wkv_ref[...], preferred_element_type=f32)
    u = ((gkv + pr2).reshape(B, _K, _W) - q2.reshape(B, 1, _W)).reshape(BK, _W)
    a = jnp.maximum(u + t1w_ref[...], lo_ref[...])

    w8 = jnp.dot(a, w1m_ref[...], preferred_element_type=f32)
    w8 = jnp.maximum(w8 * w2s_ref[...] + w2t_ref[...], 0.0)
    w8 = jnp.dot(w8, w2m_ref[...], preferred_element_type=f32)
    e = jnp.exp(w8)
    ew = jnp.dot(e, tile2_ref[...], preferred_element_type=f32)

    m = a * mv_ref[...] + mk_ref[...]
    y = m * ew
    ys = jnp.sum(y.reshape(B, _K, _W), axis=1)
    out_ref[...] = ys[:, _C:] / ys[:, :_C]


_B = 1000
_NQC = _N // _NCHUNK
_GRID = _NQC // _B


def _tc_call(chunk, g, tbl, consts):
    goff = chunk * _GRID
    full = lambda shp: pl.BlockSpec(shp, lambda i: (0, 0))
    in_specs = [
        pl.BlockSpec((_B * _K, _W), lambda i: (i, 0)),
        pl.BlockSpec((_B, _W), lambda i: (i + goff, 0)),
    ] + [full(c.shape) for c in consts]
    return pl.pallas_call(
        _tc_body,
        grid=(_GRID,),
        in_specs=in_specs,
        out_specs=pl.BlockSpec((_B, _C), lambda i: (i, 0)),
        out_shape=jax.ShapeDtypeStruct((_NQC, _C), jnp.float32),
        compiler_params=pltpu.CompilerParams(
            dimension_semantics=("arbitrary",),
            vmem_limit_bytes=120 * 1024 * 1024,
        ),
    )(g, tbl, *consts)


def kernel(p, x, idx, Wq, bq, Wk, bk, Wv, bv, p1W, p1b, pbn_g, pbn_b, pbn_m,
           pbn_v, p2W, p2b, wbn1_g, wbn1_b, wbn1_m, wbn1_v, w1W, w1b, wbn2_g,
           wbn2_b, wbn2_m, wbn2_v, w2W, w2b):
    f32 = jnp.float32
    idx32 = idx.reshape(-1).astype(jnp.int32)
    idxp = jnp.zeros((_NPAD,), jnp.int32)
    for c in range(_NCHUNK):
        idxp = idxp.at[c * _NPC : c * _NPC + _NIDX_C].set(
            idx32[c * _NIDX_C : (c + 1) * _NIDX_C])
    tbl = jnp.concatenate(
        [x, p, jnp.zeros((_N, _W - _C - 3), f32)], axis=1)

    gs = [_make_sc_gather(c)(tbl, idxp) for c in range(_NCHUNK)]

    eps = 1e-5
    s1 = wbn1_g / jnp.sqrt(wbn1_v + eps)
    t1 = wbn1_b - wbn1_m * s1
    s2 = wbn2_g / jnp.sqrt(wbn2_v + eps)
    t2 = wbn2_b - wbn2_m * s2
    ps = pbn_g / jnp.sqrt(pbn_v + eps)
    pt = pbn_b - pbn_m * ps

    def pad8(v):
        return jnp.zeros((1, 8), f32).at[0, : v.shape[0]].set(v)

    z64 = jnp.zeros((64,), f32)
    one64 = jnp.ones((64,), f32)

    wkv = (jnp.zeros((_W, _W), f32)
           .at[:_C, :_C].set(Wk.T * s1[None, :])
           .at[:_C, _C:].set(Wv.T))
    p2w = jnp.zeros((8, _W), f32).at[:3, :_C].set(p2W.T * s1[None, :]) \
                                  .at[:3, _C:].set(p2W.T)
    q2w = jnp.zeros((_W, _W), f32).at[:_C, :_C].set(Wq.T * s1[None, :])
    qb = jnp.concatenate([(bq - bk - p2b) * s1, -(bv + p2b)]).reshape(1, _W)
    t1w = jnp.concatenate([t1, z64]).reshape(1, _W)
    lo = jnp.concatenate([z64, jnp.full((64,), -jnp.inf, f32)]).reshape(1, _W)
    mv = jnp.concatenate([z64, one64]).reshape(1, _W)
    mk = jnp.concatenate([one64, z64]).reshape(1, _W)
    p1m = jnp.zeros((_W, 8), f32).at[_C:_C + 3, :3].set(p1W.T)
    pfs = pad8(ps)
    pft = pad8(p1b * ps + pt)
    w1m = jnp.zeros((_W, 8), f32).at[:_C, :].set(w1W.T)
    w2sp = pad8(s2)
    w2tp = pad8(s2 * w1b + t2)
    tile = jnp.tile(jnp.eye(_G, dtype=f32), (1, _C // _G))
    tile2 = jnp.concatenate([tile, tile], axis=1)

    consts = [
        wkv, p2w, q2w, qb, t1w, lo, mv, mk,
        p1m, pfs, pft, w1m, w2sp, w2tp, w2W.T, tile2,
    ]
    outs = [_tc_call(c, gs[c], tbl, consts) for c in range(_NCHUNK)]
    return jnp.concatenate(outs, axis=0)
